# Initial kernel scaffold; baseline (speedup 1.0000x reference)
import jax
import jax.numpy as jnp
from jax.experimental import pallas as pl


def kernel(x, edge_index, W1, b1):
    raise NotImplementedError("write your pallas kernel here")


# speedup vs baseline: 27.5129x; 27.5129x over previous
import functools

import jax
import jax.numpy as jnp
from jax import lax
from jax.experimental import pallas as pl
from jax.experimental.pallas import tpu as pltpu
from jax.experimental.pallas import tpu_sc as plsc

N = 10000
E = 320000
D = 128
NC = 2
NS = 16
NW = NC * NS
RPT = N // NS

K = 125
EW = E // NW
NCH = EW // K
DW = 8

_sc_mesh = plsc.VectorSubcoreMesh(core_axis_name="c", subcore_axis_name="s")


def _deg_body(dst_hbm, ones_hbm, zeros_hbm, degp_hbm, dstv, onesv, degsp):
    cid = lax.axis_index("c")
    sid = lax.axis_index("s")
    w = cid * NS + sid
    pltpu.sync_copy(dst_hbm.at[w], dstv)
    pltpu.sync_copy(ones_hbm, onesv)
    pltpu.sync_copy(zeros_hbm, degsp.at[pl.ds(sid * RPT, RPT)])
    plsc.subcore_barrier()

    def body(j, carry):
        pltpu.sync_copy(onesv, degsp.at[dstv.at[j]], add=True)
        return carry

    lax.fori_loop(0, NCH, body, 0)
    plsc.subcore_barrier()
    pltpu.sync_copy(degsp.at[pl.ds(sid * RPT, RPT)], degp_hbm.at[cid, sid])


_deg_call = pl.kernel(
    _deg_body,
    out_type=jax.ShapeDtypeStruct((NC, NS, RPT, DW), jnp.float32),
    mesh=_sc_mesh,
    scratch_types=[
        pltpu.VMEM((NCH, K), jnp.int32),
        pltpu.VMEM((K, DW), jnp.float32),
        pltpu.VMEM_SHARED((N, DW), jnp.float32),
    ],
)


def _scatter_body(xh_hbm, src_hbm, dst_hbm, zeros_hbm, acc_hbm,
                  srcv, dstv, rows, accsp, sem):
    cid = lax.axis_index("c")
    sid = lax.axis_index("s")
    w = cid * NS + sid
    pltpu.sync_copy(src_hbm.at[w], srcv)
    pltpu.sync_copy(dst_hbm.at[w], dstv)

    def zbody(j, carry):
        pltpu.sync_copy(zeros_hbm, accsp.at[pl.ds((sid * 5 + j) * K, K)])
        return carry

    lax.fori_loop(0, 5, zbody, 0)
    plsc.subcore_barrier()

    def body(j, carry):
        pltpu.async_copy(xh_hbm.at[srcv.at[j]], rows, sem).wait()
        pltpu.sync_copy(rows, accsp.at[dstv.at[j]], add=True)
        return carry

    lax.fori_loop(0, NCH, body, 0)
    plsc.subcore_barrier()
    pltpu.sync_copy(accsp.at[pl.ds(sid * RPT, RPT)], acc_hbm.at[cid, sid])


_scatter_call = pl.kernel(
    _scatter_body,
    out_type=jax.ShapeDtypeStruct((NC, NS, RPT, D), jnp.float32),
    mesh=_sc_mesh,
    scratch_types=[
        pltpu.VMEM((NCH, K), jnp.int32),
        pltpu.VMEM((NCH, K), jnp.int32),
        pltpu.VMEM((K, D), jnp.float32),
        pltpu.VMEM_SHARED((N, D), jnp.float32),
        pltpu.SemaphoreType.DMA,
    ],
)


BR = 1000


def _mm_body(x_ref, w_ref, degp_ref, xh_ref, dinvb_ref):
    xw = jnp.dot(x_ref[...], w_ref[...], preferred_element_type=jnp.float32)
    deg = degp_ref[:, 0] + degp_ref[:, 1] + 1.0
    dinv = lax.rsqrt(deg).reshape(BR, 1)
    xh_ref[...] = xw * dinv
    dinvb_ref[...] = jnp.broadcast_to(dinv, (BR, D))


def _mm_call(x, W1, degp_t):
    return pl.pallas_call(
        _mm_body,
        grid=(N // BR,),
        in_specs=[
            pl.BlockSpec((BR, D), lambda i: (i, 0)),
            pl.BlockSpec((D, D), lambda i: (0, 0)),
            pl.BlockSpec((BR, 2), lambda i: (i, 0)),
        ],
        out_specs=[
            pl.BlockSpec((BR, D), lambda i: (i, 0)),
            pl.BlockSpec((BR, D), lambda i: (i, 0)),
        ],
        out_shape=[
            jax.ShapeDtypeStruct((N, D), jnp.float32),
            jax.ShapeDtypeStruct((N, D), jnp.float32),
        ],
    )(x, W1, degp_t)


def _final_body(acc_ref, xh_ref, dinvb_ref, b_ref, out_ref):
    h = acc_ref[0] + acc_ref[1] + xh_ref[...]
    out_ref[...] = jnp.maximum(h * dinvb_ref[...] + b_ref[...], 0.0)


def _final_call(acc, xh, dinvb, b2):
    return pl.pallas_call(
        _final_body,
        grid=(N // BR,),
        in_specs=[
            pl.BlockSpec((NC, BR, D), lambda i: (0, i, 0)),
            pl.BlockSpec((BR, D), lambda i: (i, 0)),
            pl.BlockSpec((BR, D), lambda i: (i, 0)),
            pl.BlockSpec((1, D), lambda i: (0, 0)),
        ],
        out_specs=pl.BlockSpec((BR, D), lambda i: (i, 0)),
        out_shape=jax.ShapeDtypeStruct((N, D), jnp.float32),
    )(acc, xh, dinvb, b2)


def kernel(x, edge_index, W1, b1):
    src = edge_index[0]
    dst = edge_index[1]

    dstd = dst.reshape(NW, NCH, K)
    ones8 = jnp.ones((K, DW), jnp.float32)
    zeros8 = jnp.zeros((RPT, DW), jnp.float32)
    degp = _deg_call(dstd, ones8, zeros8).reshape(NC, N, DW)
    degp_t = jnp.stack([degp[0, :, 0], degp[1, :, 0]], axis=1)

    xh, dinvb = _mm_call(x, W1, degp_t)

    srcw = src.reshape(NW, NCH, K)
    dstw = dst.reshape(NW, NCH, K)
    zeros128 = jnp.zeros((K, D), jnp.float32)
    acc = _scatter_call(xh, srcw, dstw, zeros128).reshape(NC, N, D)

    return _final_call(acc, xh, dinvb, b1.reshape(1, D))


# speedup vs baseline: 31.4167x; 1.1419x over previous
import jax
import jax.numpy as jnp
from jax import lax
from jax.experimental import pallas as pl
from jax.experimental.pallas import tpu as pltpu
from jax.experimental.pallas import tpu_sc as plsc

N = 10000
E = 320000
D = 128
NC = 2
NS = 16
NW = NC * NS
RPT = N // NS

EW = E // NW
K = 125
NCH = EW // K
PH = 2
CPP = NCH // PH
KD = 125
NCHD = EW // KD
ZR = 125
DW = 8

_sc_mesh = plsc.VectorSubcoreMesh(core_axis_name="c", subcore_axis_name="s")


def _deg_body(dst_hbm, ones_hbm, zeros_hbm, degp_hbm, dstv, onesv, degsp, sem):
    cid = lax.axis_index("c")
    sid = lax.axis_index("s")
    w = cid * NS + sid
    pltpu.sync_copy(dst_hbm.at[w], dstv)
    pltpu.sync_copy(ones_hbm, onesv)
    pltpu.sync_copy(zeros_hbm, degsp.at[pl.ds(sid * RPT, RPT)])
    plsc.subcore_barrier()

    def fire(j, carry):
        pltpu.sync_copy(onesv, degsp.at[dstv.at[j]], add=True)
        return carry

    lax.fori_loop(0, NCHD, fire, 0)
    plsc.subcore_barrier()
    pltpu.sync_copy(degsp.at[pl.ds(sid * RPT, RPT)], degp_hbm.at[cid, sid])


_deg_call = pl.kernel(
    _deg_body,
    out_type=jax.ShapeDtypeStruct((NC, NS, RPT, DW), jnp.float32),
    mesh=_sc_mesh,
    scratch_types=[
        pltpu.VMEM((NCHD, KD), jnp.int32),
        pltpu.VMEM((KD, DW), jnp.float32),
        pltpu.VMEM_SHARED((N, DW), jnp.float32),
        pltpu.SemaphoreType.DMA,
    ],
)


def _scatter_body(xh_hbm, src_hbm, dst_hbm, zeros_hbm, acc_hbm,
                  srcv, dstv, rows0, rows1, accsp, gsem):
    cid = lax.axis_index("c")
    sid = lax.axis_index("s")
    w = cid * NS + sid

    def zbody(j, carry):
        pltpu.sync_copy(zeros_hbm, accsp.at[pl.ds((sid * 5 + j) * ZR, ZR)])
        return carry

    lax.fori_loop(0, 5, zbody, 0)
    plsc.subcore_barrier()

    bufs = (rows0, rows1)
    for p in range(PH):
        pltpu.sync_copy(src_hbm.at[w, pl.ds(p * CPP, CPP)], srcv)
        pltpu.sync_copy(dst_hbm.at[w, pl.ds(p * CPP, CPP)], dstv)
        pltpu.async_copy(xh_hbm.at[srcv.at[0]], rows0, gsem[0])

        def body(g, carry):
            for b in range(2):
                jj = g * 2 + b
                pltpu.make_async_copy(xh_hbm.at[srcv.at[0]], bufs[b],
                                      gsem[b]).wait()

                @pl.when(jj + 1 < CPP)
                def _():
                    pltpu.async_copy(xh_hbm.at[srcv.at[jj + 1]], bufs[1 - b],
                                     gsem[1 - b])

                pltpu.sync_copy(bufs[b], accsp.at[dstv.at[jj]], add=True)
            return carry

        lax.fori_loop(0, CPP // 2, body, 0)

    plsc.subcore_barrier()
    pltpu.sync_copy(accsp.at[pl.ds(sid * RPT, RPT)], acc_hbm.at[cid, sid])


_scatter_call = pl.kernel(
    _scatter_body,
    out_type=jax.ShapeDtypeStruct((NC, NS, RPT, D), jnp.float32),
    mesh=_sc_mesh,
    scratch_types=[
        pltpu.VMEM((CPP, K), jnp.int32),
        pltpu.VMEM((CPP, K), jnp.int32),
        pltpu.VMEM((K, D), jnp.float32),
        pltpu.VMEM((K, D), jnp.float32),
        pltpu.VMEM_SHARED((N, D), jnp.float32),
        [pltpu.SemaphoreType.DMA] * 2,
    ],
)


BR = 1000


def _mm_body(x_ref, w_ref, degp_ref, xh_ref):
    xw = jnp.dot(x_ref[...], w_ref[...], preferred_element_type=jnp.float32)
    deg = degp_ref[0, :, 0] + degp_ref[1, :, 0] + 1.0
    xh_ref[...] = xw * lax.rsqrt(deg).reshape(BR, 1)


def _mm_call(x, W1, degp):
    return pl.pallas_call(
        _mm_body,
        grid=(N // BR,),
        in_specs=[
            pl.BlockSpec((BR, D), lambda i: (i, 0)),
            pl.BlockSpec((D, D), lambda i: (0, 0)),
            pl.BlockSpec((NC, BR, DW), lambda i: (0, i, 0)),
        ],
        out_specs=pl.BlockSpec((BR, D), lambda i: (i, 0)),
        out_shape=jax.ShapeDtypeStruct((N, D), jnp.float32),
    )(x, W1, degp)


def _final_body(acc_ref, xh_ref, degp_ref, b_ref, out_ref):
    h = acc_ref[0] + acc_ref[1] + xh_ref[...]
    deg = degp_ref[0, :, 0] + degp_ref[1, :, 0] + 1.0
    dinv = lax.rsqrt(deg).reshape(BR, 1)
    out_ref[...] = jnp.maximum(h * dinv + b_ref[...], 0.0)


def _final_call(acc, xh, degp, b2):
    return pl.pallas_call(
        _final_body,
        grid=(N // BR,),
        in_specs=[
            pl.BlockSpec((NC, BR, D), lambda i: (0, i, 0)),
            pl.BlockSpec((BR, D), lambda i: (i, 0)),
            pl.BlockSpec((NC, BR, DW), lambda i: (0, i, 0)),
            pl.BlockSpec((1, D), lambda i: (0, 0)),
        ],
        out_specs=pl.BlockSpec((BR, D), lambda i: (i, 0)),
        out_shape=jax.ShapeDtypeStruct((N, D), jnp.float32),
    )(acc, xh, degp, b2)


def kernel(x, edge_index, W1, b1):
    src = edge_index[0]
    dst = edge_index[1]

    dstd = dst.reshape(NW, NCHD, KD)
    ones8 = jnp.ones((KD, DW), jnp.float32)
    zeros8 = jnp.zeros((RPT, DW), jnp.float32)
    degp = _deg_call(dstd, ones8, zeros8).reshape(NC, N, DW)

    xh = _mm_call(x, W1, degp)

    srcw = src.reshape(NW, NCH, K)
    dstw = dst.reshape(NW, NCH, K)
    zeros128 = jnp.zeros((ZR, D), jnp.float32)
    acc = _scatter_call(xh, srcw, dstw, zeros128).reshape(NC, N, D)

    return _final_call(acc, xh, degp, b1.reshape(1, D))
